# baseline (speedup 1.0000x reference)
import functools

import jax
import jax.numpy as jnp
from jax import lax
from jax.experimental import pallas as pl
from jax.experimental.pallas import tpu as pltpu
from jax.experimental.pallas import tpu_sc as plsc

_D = 128


def _linear_body(x_ref, wt_ref, b_ref, o_ref):
    o_ref[...] = (
        jnp.dot(x_ref[...], wt_ref[...], preferred_element_type=jnp.float32)
        + b_ref[...]
    )


def _table_linear(emb, wt, b):
    n = emb.shape[0]
    rows = 2000
    assert n % rows == 0
    return pl.pallas_call(
        _linear_body,
        grid=(n // rows,),
        in_specs=[
            pl.BlockSpec((rows, _D), lambda i: (i, 0)),
            pl.BlockSpec((_D, _D), lambda i: (0, 0)),
            pl.BlockSpec((1, _D), lambda i: (0, 0)),
        ],
        out_specs=pl.BlockSpec((rows, _D), lambda i: (i, 0)),
        out_shape=jax.ShapeDtypeStruct((n, _D), jnp.float32),
    )(emb, wt, b.reshape(1, _D))


_NBUF = 5


def _sc_gather(table, idx2d):
    nchunks, c = idx2d.shape
    total = nchunks * c
    info = plsc.get_sparse_core_info()
    nw = info.num_cores * info.num_subcores
    per_w = nchunks // nw
    assert per_w * nw == nchunks and per_w % _NBUF == 0
    groups = per_w // _NBUF

    mesh = plsc.VectorSubcoreMesh(core_axis_name="c", subcore_axis_name="s")

    @functools.partial(
        pl.kernel,
        mesh=mesh,
        out_type=jax.ShapeDtypeStruct((total, _D), jnp.float32),
        scratch_types=[
            pltpu.VMEM((per_w, c), jnp.int32),
            pltpu.VMEM((_NBUF, c, _D), jnp.float32),
        ]
        + [pltpu.SemaphoreType.DMA] * (2 * _NBUF),
    )
    def k(table_hbm, idx_hbm, out_hbm, idx_v, rows_v, *sems):
        gsems, osems = sems[:_NBUF], sems[_NBUF:]
        wid = lax.axis_index("s") * info.num_cores + lax.axis_index("c")
        cbase = wid * per_w
        pltpu.sync_copy(idx_hbm.at[pl.ds(cbase, per_w)], idx_v)

        def gather(j, s):
            return pltpu.make_async_copy(
                table_hbm.at[idx_v.at[j]], rows_v.at[s], gsems[s]
            )

        def writeback(j, s):
            return pltpu.make_async_copy(
                rows_v.at[s], out_hbm.at[pl.ds((cbase + j) * c, c)], osems[s]
            )

        for s in range(_NBUF):
            gather(s, s).start()

        def group_body(g, carry):
            j0 = g * _NBUF
            for s in range(_NBUF):
                gather(j0 + s, s).wait()
                writeback(j0 + s, s).start()
            for s in range(_NBUF):
                writeback(j0 + s, s).wait()
                gather(j0 + _NBUF + s, s).start()
            return carry

        lax.fori_loop(0, groups - 1, group_body, 0)

        j0 = (groups - 1) * _NBUF
        for s in range(_NBUF):
            gather(j0 + s, s).wait()
            writeback(j0 + s, s).start()
        for s in range(_NBUF):
            writeback(j0 + s, s).wait()

    return k(table, idx2d)


def kernel(item_seq, emb_table, W, b):
    bsz, seq = item_seq.shape
    e2 = _table_linear(emb_table, W.T, b)
    idx2d = item_seq.reshape(-1, 128).astype(jnp.int32)
    out = _sc_gather(e2, idx2d)
    return out.reshape(bsz, seq, _D)


# speedup vs baseline: 1.0507x; 1.0507x over previous
import functools

import jax
import jax.numpy as jnp
from jax import lax
from jax.experimental import pallas as pl
from jax.experimental.pallas import tpu as pltpu
from jax.experimental.pallas import tpu_sc as plsc

_D = 128


def _linear_body(x_ref, w_ref, b_ref, o_ref):
    o_ref[...] = (
        lax.dot_general(
            x_ref[...], w_ref[...], (((1,), (1,)), ((), ())),
            preferred_element_type=jnp.float32,
        )
        + b_ref[...]
    )


def _table_linear(emb, w, b):
    n = emb.shape[0]
    rows = 4000
    assert n % rows == 0
    return pl.pallas_call(
        _linear_body,
        grid=(n // rows,),
        in_specs=[
            pl.BlockSpec((rows, _D), lambda i: (i, 0)),
            pl.BlockSpec((_D, _D), lambda i: (0, 0)),
            pl.BlockSpec((1, _D), lambda i: (0, 0)),
        ],
        out_specs=pl.BlockSpec((rows, _D), lambda i: (i, 0)),
        out_shape=jax.ShapeDtypeStruct((n, _D), jnp.float32),
    )(emb, w, b.reshape(1, _D))


_NBUF = 4


def _sc_gather(table, idx2d):
    nchunks, c = idx2d.shape
    total = nchunks * c
    info = plsc.get_sparse_core_info()
    nw = info.num_cores * info.num_subcores
    per_w = nchunks // nw
    assert per_w * nw == nchunks and per_w % _NBUF == 0
    groups = per_w // _NBUF
    idx3 = idx2d.reshape(nw, per_w, c)

    mesh = plsc.VectorSubcoreMesh(core_axis_name="c", subcore_axis_name="s")

    @functools.partial(
        pl.kernel,
        mesh=mesh,
        out_type=jax.ShapeDtypeStruct((total, _D), jnp.float32),
        scratch_types=[
            pltpu.VMEM((per_w, c), jnp.int32),
            pltpu.VMEM((_NBUF, c, _D), jnp.float32),
        ]
        + [pltpu.SemaphoreType.DMA] * (2 * _NBUF),
    )
    def k(table_hbm, idx_hbm, out_hbm, idx_v, rows_v, *sems):
        gsems, osems = sems[:_NBUF], sems[_NBUF:]
        wid = lax.axis_index("s") * info.num_cores + lax.axis_index("c")
        cbase = wid * per_w
        pltpu.sync_copy(idx_hbm.at[wid], idx_v)

        def gather(j, s):
            return pltpu.make_async_copy(
                table_hbm.at[idx_v.at[j]], rows_v.at[s], gsems[s]
            )

        def writeback(j, s):
            return pltpu.make_async_copy(
                rows_v.at[s], out_hbm.at[pl.ds((cbase + j) * c, c)], osems[s]
            )

        for s in range(_NBUF):
            gather(s, s).start()

        def group_body(g, carry):
            j0 = g * _NBUF
            for s in range(_NBUF):
                gather(j0 + s, s).wait()
                writeback(j0 + s, s).start()
            for s in range(_NBUF):
                writeback(j0 + s, s).wait()
                gather(j0 + _NBUF + s, s).start()
            return carry

        lax.fori_loop(0, groups - 1, group_body, 0)

        j0 = (groups - 1) * _NBUF
        for s in range(_NBUF):
            gather(j0 + s, s).wait()
            writeback(j0 + s, s).start()
        for s in range(_NBUF):
            writeback(j0 + s, s).wait()

    return k(table, idx3)


def kernel(item_seq, emb_table, W, b):
    bsz, seq = item_seq.shape
    e2 = _table_linear(emb_table, W, b)
    idx2d = item_seq.reshape(-1, 128).astype(jnp.int32)
    out = _sc_gather(e2, idx2d)
    return out.reshape(bsz, seq, _D)


# speedup vs baseline: 1.0657x; 1.0143x over previous
import functools

import jax
import jax.numpy as jnp
from jax import lax
from jax.experimental import pallas as pl
from jax.experimental.pallas import tpu as pltpu
from jax.experimental.pallas import tpu_sc as plsc

_D = 128


def _linear_body(x_ref, w_ref, b_ref, o_ref):
    o_ref[...] = (
        lax.dot_general(
            x_ref[...], w_ref[...], (((1,), (1,)), ((), ())),
            preferred_element_type=jnp.float32,
        )
        + b_ref[...]
    )


def _table_linear(emb, w, b):
    n = emb.shape[0]
    rows = 10000
    assert n % rows == 0
    return pl.pallas_call(
        _linear_body,
        grid=(n // rows,),
        in_specs=[
            pl.BlockSpec((rows, _D), lambda i: (i, 0)),
            pl.BlockSpec((_D, _D), lambda i: (0, 0)),
            pl.BlockSpec((1, _D), lambda i: (0, 0)),
        ],
        out_specs=pl.BlockSpec((rows, _D), lambda i: (i, 0)),
        out_shape=jax.ShapeDtypeStruct((n, _D), jnp.float32),
    )(emb, w, b.reshape(1, _D))


_NBUF = 4


def _sc_gather(table, idx2d):
    nchunks, c = idx2d.shape
    total = nchunks * c
    info = plsc.get_sparse_core_info()
    nw = info.num_cores * info.num_subcores
    per_w = nchunks // nw
    assert per_w * nw == nchunks and per_w % _NBUF == 0
    groups = per_w // _NBUF
    idx3 = idx2d.reshape(nw, per_w, c)

    mesh = plsc.VectorSubcoreMesh(core_axis_name="c", subcore_axis_name="s")

    @functools.partial(
        pl.kernel,
        mesh=mesh,
        out_type=jax.ShapeDtypeStruct((total, _D), jnp.float32),
        scratch_types=[
            pltpu.VMEM((per_w, c), jnp.int32),
            pltpu.VMEM((_NBUF, c, _D), jnp.float32),
        ]
        + [pltpu.SemaphoreType.DMA] * (2 * _NBUF),
    )
    def k(table_hbm, idx_hbm, out_hbm, idx_v, rows_v, *sems):
        gsems, osems = sems[:_NBUF], sems[_NBUF:]
        wid = lax.axis_index("s") * info.num_cores + lax.axis_index("c")
        cbase = wid * per_w
        pltpu.sync_copy(idx_hbm.at[wid], idx_v)

        def gather(j, s):
            return pltpu.make_async_copy(
                table_hbm.at[idx_v.at[j]], rows_v.at[s], gsems[s]
            )

        def writeback(j, s):
            return pltpu.make_async_copy(
                rows_v.at[s], out_hbm.at[pl.ds((cbase + j) * c, c)], osems[s]
            )

        for s in range(_NBUF):
            gather(s, s).start()

        def group_body(g, carry):
            j0 = g * _NBUF
            for s in range(_NBUF):
                gather(j0 + s, s).wait()
                writeback(j0 + s, s).start()
            for s in range(_NBUF):
                writeback(j0 + s, s).wait()
                gather(j0 + _NBUF + s, s).start()
            return carry

        lax.fori_loop(0, groups - 1, group_body, 0)

        j0 = (groups - 1) * _NBUF
        for s in range(_NBUF):
            gather(j0 + s, s).wait()
            writeback(j0 + s, s).start()
        for s in range(_NBUF):
            writeback(j0 + s, s).wait()

    return k(table, idx3)


def kernel(item_seq, emb_table, W, b):
    bsz, seq = item_seq.shape
    e2 = _table_linear(emb_table, W, b)
    idx2d = item_seq.reshape(-1, 128).astype(jnp.int32)
    out = _sc_gather(e2, idx2d)
    return out.reshape(bsz, seq, _D)


# speedup vs baseline: 1.0723x; 1.0062x over previous
import functools

import jax
import jax.numpy as jnp
from jax import lax
from jax.experimental import pallas as pl
from jax.experimental.pallas import tpu as pltpu
from jax.experimental.pallas import tpu_sc as plsc

_D = 128


def _linear_body(x_ref, w_ref, b_ref, o_ref):
    o_ref[...] = (
        lax.dot_general(
            x_ref[...], w_ref[...], (((1,), (1,)), ((), ())),
            preferred_element_type=jnp.float32,
        )
        + b_ref[...]
    )


def _table_linear(emb, w, b):
    n = emb.shape[0]
    rows = 20000
    assert n % rows == 0
    return pl.pallas_call(
        _linear_body,
        grid=(n // rows,),
        in_specs=[
            pl.BlockSpec((rows, _D), lambda i: (i, 0)),
            pl.BlockSpec((_D, _D), lambda i: (0, 0)),
            pl.BlockSpec((1, _D), lambda i: (0, 0)),
        ],
        out_specs=pl.BlockSpec((rows, _D), lambda i: (i, 0)),
        out_shape=jax.ShapeDtypeStruct((n, _D), jnp.float32),
    )(emb, w, b.reshape(1, _D))


_NBUF = 4


def _sc_gather(table, idx2d):
    nchunks, c = idx2d.shape
    total = nchunks * c
    info = plsc.get_sparse_core_info()
    nw = info.num_cores * info.num_subcores
    per_w = nchunks // nw
    assert per_w * nw == nchunks and per_w % _NBUF == 0
    groups = per_w // _NBUF
    idx3 = idx2d.reshape(nw, per_w, c)

    mesh = plsc.VectorSubcoreMesh(core_axis_name="c", subcore_axis_name="s")

    @functools.partial(
        pl.kernel,
        mesh=mesh,
        out_type=jax.ShapeDtypeStruct((total, _D), jnp.float32),
        scratch_types=[
            pltpu.VMEM((per_w, c), jnp.int32),
            pltpu.VMEM((_NBUF, c, _D), jnp.float32),
        ]
        + [pltpu.SemaphoreType.DMA] * (2 * _NBUF),
    )
    def k(table_hbm, idx_hbm, out_hbm, idx_v, rows_v, *sems):
        gsems, osems = sems[:_NBUF], sems[_NBUF:]
        wid = lax.axis_index("s") * info.num_cores + lax.axis_index("c")
        cbase = wid * per_w
        pltpu.sync_copy(idx_hbm.at[wid], idx_v)

        def gather(j, s):
            return pltpu.make_async_copy(
                table_hbm.at[idx_v.at[j]], rows_v.at[s], gsems[s]
            )

        def writeback(j, s):
            return pltpu.make_async_copy(
                rows_v.at[s], out_hbm.at[pl.ds((cbase + j) * c, c)], osems[s]
            )

        for s in range(_NBUF):
            gather(s, s).start()

        def group_body(g, carry):
            j0 = g * _NBUF
            for s in range(_NBUF):
                gather(j0 + s, s).wait()
                writeback(j0 + s, s).start()
            for s in range(_NBUF):
                writeback(j0 + s, s).wait()
                gather(j0 + _NBUF + s, s).start()
            return carry

        lax.fori_loop(0, groups - 1, group_body, 0)

        j0 = (groups - 1) * _NBUF
        for s in range(_NBUF):
            gather(j0 + s, s).wait()
            writeback(j0 + s, s).start()
        for s in range(_NBUF):
            writeback(j0 + s, s).wait()

    return k(table, idx3)


def kernel(item_seq, emb_table, W, b):
    bsz, seq = item_seq.shape
    e2 = _table_linear(emb_table, W, b)
    idx2d = item_seq.reshape(-1, 128).astype(jnp.int32)
    out = _sc_gather(e2, idx2d)
    return out.reshape(bsz, seq, _D)


# speedup vs baseline: 1.0732x; 1.0008x over previous
import functools

import jax
import jax.numpy as jnp
from jax import lax
from jax.experimental import pallas as pl
from jax.experimental.pallas import tpu as pltpu
from jax.experimental.pallas import tpu_sc as plsc

_D = 128


def _linear_body(x_ref, w_ref, b_ref, o_ref):
    o_ref[...] = (
        lax.dot_general(
            x_ref[...], w_ref[...], (((1,), (1,)), ((), ())),
            preferred_element_type=jnp.float32,
        )
        + b_ref[...]
    )


def _table_linear(emb, w, b):
    n = emb.shape[0]
    rows = 20000
    assert n % rows == 0
    return pl.pallas_call(
        _linear_body,
        grid=(n // rows,),
        in_specs=[
            pl.BlockSpec((rows, _D), lambda i: (i, 0)),
            pl.BlockSpec((_D, _D), lambda i: (0, 0)),
            pl.BlockSpec((1, _D), lambda i: (0, 0)),
        ],
        out_specs=pl.BlockSpec((rows, _D), lambda i: (i, 0)),
        out_shape=jax.ShapeDtypeStruct((n, _D), jnp.float32),
    )(emb, w, b.reshape(1, _D))


_NBUF = 8


def _sc_gather(table, idx2d):
    nchunks, c = idx2d.shape
    total = nchunks * c
    info = plsc.get_sparse_core_info()
    nw = info.num_cores * info.num_subcores
    per_w = nchunks // nw
    assert per_w * nw == nchunks and per_w % _NBUF == 0
    groups = per_w // _NBUF
    idx3 = idx2d.reshape(nw, per_w, c)

    mesh = plsc.VectorSubcoreMesh(core_axis_name="c", subcore_axis_name="s")

    @functools.partial(
        pl.kernel,
        mesh=mesh,
        out_type=jax.ShapeDtypeStruct((total, _D), jnp.float32),
        scratch_types=[
            pltpu.VMEM((per_w, c), jnp.int32),
            pltpu.VMEM((_NBUF, c, _D), jnp.float32),
        ]
        + [pltpu.SemaphoreType.DMA] * (2 * _NBUF),
    )
    def k(table_hbm, idx_hbm, out_hbm, idx_v, rows_v, *sems):
        gsems, osems = sems[:_NBUF], sems[_NBUF:]
        wid = lax.axis_index("s") * info.num_cores + lax.axis_index("c")
        cbase = wid * per_w
        pltpu.sync_copy(idx_hbm.at[wid], idx_v)

        def gather(j, s):
            return pltpu.make_async_copy(
                table_hbm.at[idx_v.at[j]], rows_v.at[s], gsems[s]
            )

        def writeback(j, s):
            return pltpu.make_async_copy(
                rows_v.at[s], out_hbm.at[pl.ds((cbase + j) * c, c)], osems[s]
            )

        for s in range(_NBUF):
            gather(s, s).start()

        def group_body(g, carry):
            j0 = g * _NBUF
            for s in range(_NBUF):
                gather(j0 + s, s).wait()
                writeback(j0 + s, s).start()
            for s in range(_NBUF):
                writeback(j0 + s, s).wait()
                gather(j0 + _NBUF + s, s).start()
            return carry

        lax.fori_loop(0, groups - 1, group_body, 0)

        j0 = (groups - 1) * _NBUF
        for s in range(_NBUF):
            gather(j0 + s, s).wait()
            writeback(j0 + s, s).start()
        for s in range(_NBUF):
            writeback(j0 + s, s).wait()

    return k(table, idx3)


def kernel(item_seq, emb_table, W, b):
    bsz, seq = item_seq.shape
    e2 = _table_linear(emb_table, W, b)
    idx2d = item_seq.reshape(-1, 64).astype(jnp.int32)
    out = _sc_gather(e2, idx2d)
    return out.reshape(bsz, seq, _D)
